# baseline (speedup 1.0000x reference)
import functools

import jax
import jax.numpy as jnp
from jax import lax
from jax.experimental import pallas as pl
from jax.experimental.pallas import tpu as pltpu
from jax.experimental.pallas import tpu_sc as plsc

N_USERS = 6000
N_ITEMS = 4000
N = N_USERS + N_ITEMS
E = 320000
D = 128

NC = 2
NS = 16
NW = NC * NS

CH = 128
G = 256
EW = 10240
HW = EW // 2
K_DEG = -(-(E // NS) // CH)
ET_PAD = K_DEG * CH

N_ACC = 10240
ROWS_T = N_ACC // NS
DUMP = N

_mesh = plsc.VectorSubcoreMesh(core_axis_name="c", subcore_axis_name="s")


def _zero_fill(zbuf, rows, width):
    zero = jnp.zeros((16,), jnp.float32)
    for i in range(rows):
        for l in range(width // 16):
            zbuf[i, pl.ds(l * 16, 16)] = zero


@functools.partial(
    pl.kernel,
    out_type=jax.ShapeDtypeStruct((NC, N_ACC, D), jnp.float32),
    mesh=_mesh,
    scratch_types=[
        pltpu.VMEM((K_DEG, CH), jnp.int32),
        pltpu.VMEM((CH, D), jnp.float32),
        pltpu.VMEM((64, D), jnp.float32),
        pltpu.VMEM_SHARED((N_ACC, D), jnp.float32),
    ],
)
def _deg_kernel(idx_hbm, out_hbm, idx_v, ones_v, zbuf, acc_sh):
    c = lax.axis_index("c")
    s = lax.axis_index("s")
    one = jnp.full((16,), 1.0, jnp.float32)
    for i in range(CH):
        for l in range(D // 16):
            ones_v[i, pl.ds(l * 16, 16)] = one
    _zero_fill(zbuf, 64, D)
    for t in range(ROWS_T // 64):
        pltpu.sync_copy(zbuf, acc_sh.at[pl.ds(s * ROWS_T + t * 64, 64)])
    pltpu.sync_copy(idx_hbm.at[c * NS + s], idx_v)
    plsc.subcore_barrier()

    def step(j, carry):
        pltpu.sync_copy(ones_v, acc_sh.at[idx_v.at[j]], add=True)
        return carry

    lax.fori_loop(0, K_DEG, step, 0)
    plsc.subcore_barrier()
    pltpu.sync_copy(acc_sh.at[pl.ds(s * ROWS_T, ROWS_T)],
                    out_hbm.at[c, pl.ds(s * ROWS_T, ROWS_T)])


@functools.partial(
    pl.kernel,
    out_type=jax.ShapeDtypeStruct((NC, N_ACC, D), jnp.float32),
    mesh=_mesh,
    scratch_types=[
        pltpu.VMEM((HW,), jnp.int32),
        pltpu.VMEM((HW,), jnp.int32),
        pltpu.VMEM((G, D), jnp.float32),
        pltpu.VMEM_SHARED((N_ACC, D), jnp.float32),
        pltpu.SemaphoreType.DMA,
    ],
)
def _push_kernel(src_hbm, dst_hbm, table_hbm, out_hbm,
                 src_v, dst_v, rows, acc_sh, sem):
    c = lax.axis_index("c")
    s = lax.axis_index("s")
    wid = c * NS + s
    _zero_fill(rows, CH, D)
    for t in range(ROWS_T // CH):
        pltpu.sync_copy(rows.at[pl.ds(0, CH)],
                        acc_sh.at[pl.ds(s * ROWS_T + t * CH, CH)])
    plsc.subcore_barrier()
    for h in (0, 1):
        pltpu.sync_copy(src_hbm.at[wid, pl.ds(h * HW, HW)], src_v)
        pltpu.sync_copy(dst_hbm.at[wid, pl.ds(h * HW, HW)], dst_v)

        def step(u, carry):
            pltpu.async_copy(table_hbm.at[src_v.at[pl.ds(u * G, G)]],
                             rows, sem).wait()
            pltpu.sync_copy(rows, acc_sh.at[dst_v.at[pl.ds(u * G, G)]], add=True)
            return carry

        lax.fori_loop(0, HW // G, step, 0)
    plsc.subcore_barrier()
    pltpu.sync_copy(acc_sh.at[pl.ds(s * ROWS_T, ROWS_T)],
                    out_hbm.at[c, pl.ds(s * ROWS_T, ROWS_T)])


def _prep_body(all_ref, dego_ref, out_ref):
    rs = lax.rsqrt(jnp.maximum(dego_ref[...], 1.0))
    out_ref[...] = all_ref[...] * rs


BR = 1000


def _prep_call(all0, deg_out):
    return pl.pallas_call(
        _prep_body,
        out_shape=jax.ShapeDtypeStruct((N, D), jnp.float32),
        grid=(N // BR,),
        in_specs=[
            pl.BlockSpec((BR, D), lambda i: (i, 0)),
            pl.BlockSpec((BR, 1), lambda i: (i, 0)),
        ],
        out_specs=pl.BlockSpec((BR, D), lambda i: (i, 0)),
    )(all0, deg_out)


def _layer_body(acc_ref, all_ref, degi_ref, dego_ref,
                w1_ref, b1_ref, w2_ref, b2_ref,
                raw_ref, norm_ref, scaled_ref):
    nei = (acc_ref[0] + acc_ref[1]) * lax.rsqrt(jnp.maximum(degi_ref[...], 1.0))
    alle = all_ref[...]
    w1b = w1_ref[...] + b1_ref[...]
    w2b = w2_ref[...] + b2_ref[...]
    sum_e = jnp.dot(nei + alle, w1b, preferred_element_type=jnp.float32)
    sum_e = jnp.where(sum_e >= 0, sum_e, 0.2 * sum_e)
    bi = jnp.dot(nei * alle, w2b, preferred_element_type=jnp.float32)
    bi = jnp.where(bi >= 0, bi, 0.2 * bi)
    new = sum_e + bi
    nrm = jnp.sqrt(jnp.sum(new * new, axis=1, keepdims=True))
    raw_ref[...] = new
    norm_ref[...] = new / jnp.maximum(nrm, 1e-12)
    scaled_ref[...] = new * lax.rsqrt(jnp.maximum(dego_ref[...], 1.0))


def _layer_call(acc, alle, deg_in, deg_out, W1, b1, W2, b2):
    return pl.pallas_call(
        _layer_body,
        out_shape=(
            jax.ShapeDtypeStruct((N, D), jnp.float32),
            jax.ShapeDtypeStruct((N, D), jnp.float32),
            jax.ShapeDtypeStruct((N, D), jnp.float32),
        ),
        grid=(N // BR,),
        in_specs=[
            pl.BlockSpec((NC, BR, D), lambda i: (0, i, 0)),
            pl.BlockSpec((BR, D), lambda i: (i, 0)),
            pl.BlockSpec((BR, 1), lambda i: (i, 0)),
            pl.BlockSpec((BR, 1), lambda i: (i, 0)),
            pl.BlockSpec((D, D), lambda i: (0, 0)),
            pl.BlockSpec((1, D), lambda i: (0, 0)),
            pl.BlockSpec((D, D), lambda i: (0, 0)),
            pl.BlockSpec((1, D), lambda i: (0, 0)),
        ],
        out_specs=(
            pl.BlockSpec((BR, D), lambda i: (i, 0)),
            pl.BlockSpec((BR, D), lambda i: (i, 0)),
            pl.BlockSpec((BR, D), lambda i: (i, 0)),
        ),
    )(acc, alle, deg_in, deg_out, W1, b1, W2, b2)


def kernel(edge_index, embed_user, embed_item,
           W1_0, b1_0, W2_0, b2_0, W1_1, b1_1, W2_1, b2_1):
    src = edge_index[0].astype(jnp.int32)
    dst = edge_index[1].astype(jnp.int32)
    all0 = jnp.concatenate([embed_user, embed_item], axis=0)

    pad_d = jnp.full((NS * ET_PAD - E,), DUMP, jnp.int32)
    src_d = jnp.concatenate([src, pad_d]).reshape(NS, K_DEG, CH)
    dst_d = jnp.concatenate([dst, pad_d]).reshape(NS, K_DEG, CH)
    idx_deg = jnp.concatenate([src_d, dst_d], axis=0)
    deg = _deg_kernel(idx_deg)
    deg_out = deg[0, :N, :1]
    deg_in = deg[1, :N, :1]

    scaled0 = _prep_call(all0, deg_out)

    pad_s = jnp.zeros((NW * EW - E,), jnp.int32)
    pad_t = jnp.full((NW * EW - E,), DUMP, jnp.int32)
    srcp = jnp.concatenate([src, pad_s]).reshape(NW, EW)
    dstp = jnp.concatenate([dst, pad_t]).reshape(NW, EW)

    acc1 = _push_kernel(srcp, dstp, scaled0)[:, :N]
    raw1, norm1, scaled1 = _layer_call(acc1, all0, deg_in, deg_out,
                                       W1_0, b1_0, W2_0, b2_0)
    acc2 = _push_kernel(srcp, dstp, scaled1)[:, :N]
    _, norm2, _ = _layer_call(acc2, raw1, deg_in, deg_out,
                              W1_1, b1_1, W2_1, b2_1)

    final = jnp.concatenate([all0, norm1, norm2], axis=1)
    return (final[:N_USERS], final[N_USERS:])


# speedup vs baseline: 1.4234x; 1.4234x over previous
import functools

import jax
import jax.numpy as jnp
from jax import lax
from jax.experimental import pallas as pl
from jax.experimental.pallas import tpu as pltpu
from jax.experimental.pallas import tpu_sc as plsc

N_USERS = 6000
N_ITEMS = 4000
N = N_USERS + N_ITEMS
E = 320000
D = 128

NC = 2
NS = 16
NW = NC * NS

CH = 128
K_PUSH = -(-(E // NW) // CH)
EW_PAD = K_PUSH * CH
K_DEG = -(-(E // NS) // CH)
ET_PAD = K_DEG * CH

N_ACC = 10240
ROWS_T = N_ACC // NS
DUMP = N

_mesh = plsc.VectorSubcoreMesh(core_axis_name="c", subcore_axis_name="s")


def _zero_fill(zbuf, rows, width):
    zero = jnp.zeros((16,), jnp.float32)
    for i in range(rows):
        for l in range(width // 16):
            zbuf[i, pl.ds(l * 16, 16)] = zero


@functools.partial(
    pl.kernel,
    out_type=jax.ShapeDtypeStruct((NC, N_ACC, D), jnp.float32),
    mesh=_mesh,
    scratch_types=[
        pltpu.VMEM((K_DEG, CH), jnp.int32),
        pltpu.VMEM((CH, D), jnp.float32),
        pltpu.VMEM((64, D), jnp.float32),
        pltpu.VMEM_SHARED((N_ACC, D), jnp.float32),
    ],
)
def _deg_kernel(idx_hbm, out_hbm, idx_v, ones_v, zbuf, acc_sh):
    c = lax.axis_index("c")
    s = lax.axis_index("s")
    one = jnp.full((16,), 1.0, jnp.float32)
    for i in range(CH):
        for l in range(D // 16):
            ones_v[i, pl.ds(l * 16, 16)] = one
    _zero_fill(zbuf, 64, D)
    for t in range(ROWS_T // 64):
        pltpu.sync_copy(zbuf, acc_sh.at[pl.ds(s * ROWS_T + t * 64, 64)])
    pltpu.sync_copy(idx_hbm.at[c * NS + s], idx_v)
    plsc.subcore_barrier()

    def step(j, carry):
        pltpu.sync_copy(ones_v, acc_sh.at[idx_v.at[j]], add=True)
        return carry

    lax.fori_loop(0, K_DEG, step, 0)
    plsc.subcore_barrier()
    pltpu.sync_copy(acc_sh.at[pl.ds(s * ROWS_T, ROWS_T)],
                    out_hbm.at[c, pl.ds(s * ROWS_T, ROWS_T)])


@functools.partial(
    pl.kernel,
    out_type=jax.ShapeDtypeStruct((NC, N_ACC, D), jnp.float32),
    mesh=_mesh,
    scratch_types=[
        pltpu.VMEM((K_PUSH, CH), jnp.int32),
        pltpu.VMEM((K_PUSH, CH), jnp.int32),
        pltpu.VMEM((CH, D), jnp.float32),
        pltpu.VMEM((64, D), jnp.float32),
        pltpu.VMEM_SHARED((N_ACC, D), jnp.float32),
        pltpu.SemaphoreType.DMA,
    ],
)
def _push_kernel(src_hbm, dst_hbm, table_hbm, out_hbm,
                 src_v, dst_v, row_v, zbuf, acc_sh, sem):
    c = lax.axis_index("c")
    s = lax.axis_index("s")
    wid = c * NS + s
    _zero_fill(zbuf, 64, D)
    for t in range(ROWS_T // 64):
        pltpu.sync_copy(zbuf, acc_sh.at[pl.ds(s * ROWS_T + t * 64, 64)])
    pltpu.sync_copy(src_hbm.at[wid], src_v)
    pltpu.sync_copy(dst_hbm.at[wid], dst_v)
    plsc.subcore_barrier()

    def step(j, carry):
        pltpu.async_copy(table_hbm.at[src_v.at[j]], row_v, sem).wait()
        pltpu.sync_copy(row_v, acc_sh.at[dst_v.at[j]], add=True)
        return carry

    lax.fori_loop(0, K_PUSH, step, 0)
    plsc.subcore_barrier()
    pltpu.sync_copy(acc_sh.at[pl.ds(s * ROWS_T, ROWS_T)],
                    out_hbm.at[c, pl.ds(s * ROWS_T, ROWS_T)])


def _prep_body(all_ref, dego_ref, out_ref):
    rs = lax.rsqrt(jnp.maximum(dego_ref[...], 1.0))
    out_ref[...] = all_ref[...] * rs


BR = 1000


def _prep_call(all0, deg_out):
    return pl.pallas_call(
        _prep_body,
        out_shape=jax.ShapeDtypeStruct((N, D), jnp.float32),
        grid=(N // BR,),
        in_specs=[
            pl.BlockSpec((BR, D), lambda i: (i, 0)),
            pl.BlockSpec((BR, 1), lambda i: (i, 0)),
        ],
        out_specs=pl.BlockSpec((BR, D), lambda i: (i, 0)),
    )(all0, deg_out)


def _layer_body(acc_ref, all_ref, degi_ref, dego_ref,
                w1_ref, b1_ref, w2_ref, b2_ref,
                raw_ref, norm_ref, scaled_ref):
    nei = (acc_ref[0] + acc_ref[1]) * lax.rsqrt(jnp.maximum(degi_ref[...], 1.0))
    alle = all_ref[...]
    w1b = w1_ref[...] + b1_ref[...]
    w2b = w2_ref[...] + b2_ref[...]
    sum_e = jnp.dot(nei + alle, w1b, preferred_element_type=jnp.float32)
    sum_e = jnp.where(sum_e >= 0, sum_e, 0.2 * sum_e)
    bi = jnp.dot(nei * alle, w2b, preferred_element_type=jnp.float32)
    bi = jnp.where(bi >= 0, bi, 0.2 * bi)
    new = sum_e + bi
    nrm = jnp.sqrt(jnp.sum(new * new, axis=1, keepdims=True))
    raw_ref[...] = new
    norm_ref[...] = new / jnp.maximum(nrm, 1e-12)
    scaled_ref[...] = new * lax.rsqrt(jnp.maximum(dego_ref[...], 1.0))


def _layer_call(acc, alle, deg_in, deg_out, W1, b1, W2, b2):
    return pl.pallas_call(
        _layer_body,
        out_shape=(
            jax.ShapeDtypeStruct((N, D), jnp.float32),
            jax.ShapeDtypeStruct((N, D), jnp.float32),
            jax.ShapeDtypeStruct((N, D), jnp.float32),
        ),
        grid=(N // BR,),
        in_specs=[
            pl.BlockSpec((NC, BR, D), lambda i: (0, i, 0)),
            pl.BlockSpec((BR, D), lambda i: (i, 0)),
            pl.BlockSpec((BR, 1), lambda i: (i, 0)),
            pl.BlockSpec((BR, 1), lambda i: (i, 0)),
            pl.BlockSpec((D, D), lambda i: (0, 0)),
            pl.BlockSpec((1, D), lambda i: (0, 0)),
            pl.BlockSpec((D, D), lambda i: (0, 0)),
            pl.BlockSpec((1, D), lambda i: (0, 0)),
        ],
        out_specs=(
            pl.BlockSpec((BR, D), lambda i: (i, 0)),
            pl.BlockSpec((BR, D), lambda i: (i, 0)),
            pl.BlockSpec((BR, D), lambda i: (i, 0)),
        ),
    )(acc, alle, deg_in, deg_out, W1, b1, W2, b2)


def kernel(edge_index, embed_user, embed_item,
           W1_0, b1_0, W2_0, b2_0, W1_1, b1_1, W2_1, b2_1):
    src = edge_index[0].astype(jnp.int32)
    dst = edge_index[1].astype(jnp.int32)
    all0 = jnp.concatenate([embed_user, embed_item], axis=0)

    pad_d = jnp.full((NS * ET_PAD - E,), DUMP, jnp.int32)
    src_d = jnp.concatenate([src, pad_d]).reshape(NS, K_DEG, CH)
    dst_d = jnp.concatenate([dst, pad_d]).reshape(NS, K_DEG, CH)
    idx_deg = jnp.concatenate([src_d, dst_d], axis=0)
    deg = _deg_kernel(idx_deg)
    deg_out = deg[0, :N, :1]
    deg_in = deg[1, :N, :1]

    scaled0 = _prep_call(all0, deg_out)

    pad_s = jnp.zeros((NW * EW_PAD - E,), jnp.int32)
    pad_t = jnp.full((NW * EW_PAD - E,), DUMP, jnp.int32)
    srcp = jnp.concatenate([src, pad_s]).reshape(NW, K_PUSH, CH)
    dstp = jnp.concatenate([dst, pad_t]).reshape(NW, K_PUSH, CH)

    acc1 = _push_kernel(srcp, dstp, scaled0)[:, :N]
    raw1, norm1, scaled1 = _layer_call(acc1, all0, deg_in, deg_out,
                                       W1_0, b1_0, W2_0, b2_0)
    acc2 = _push_kernel(srcp, dstp, scaled1)[:, :N]
    _, norm2, _ = _layer_call(acc2, raw1, deg_in, deg_out,
                              W1_1, b1_1, W2_1, b2_1)

    final = jnp.concatenate([all0, norm1, norm2], axis=1)
    return (final[:N_USERS], final[N_USERS:])


# speedup vs baseline: 1.5793x; 1.1095x over previous
import functools

import jax
import jax.numpy as jnp
from jax import lax
from jax.experimental import pallas as pl
from jax.experimental.pallas import tpu as pltpu
from jax.experimental.pallas import tpu_sc as plsc

N_USERS = 6000
N_ITEMS = 4000
N = N_USERS + N_ITEMS
E = 320000
D = 128

NC = 2
NS = 16
NW = NC * NS

CH = 128
K_PUSH = -(-(E // NW) // CH)
EW_PAD = K_PUSH * CH
K_DEG = -(-(E // NS) // CH)
ET_PAD = K_DEG * CH

N_ACC = 10240
ROWS_T = N_ACC // NS
DUMP = N

_mesh = plsc.VectorSubcoreMesh(core_axis_name="c", subcore_axis_name="s")


def _zero_fill(zbuf, rows, width):
    zero = jnp.zeros((16,), jnp.float32)
    for i in range(rows):
        for l in range(width // 16):
            zbuf[i, pl.ds(l * 16, 16)] = zero


@functools.partial(
    pl.kernel,
    out_type=jax.ShapeDtypeStruct((NC, N_ACC, D), jnp.float32),
    mesh=_mesh,
    scratch_types=[
        pltpu.VMEM((K_DEG, CH), jnp.int32),
        pltpu.VMEM((CH, D), jnp.float32),
        pltpu.VMEM((64, D), jnp.float32),
        pltpu.VMEM_SHARED((N_ACC, D), jnp.float32),
    ],
)
def _deg_kernel(idx_hbm, out_hbm, idx_v, ones_v, zbuf, acc_sh):
    c = lax.axis_index("c")
    s = lax.axis_index("s")
    one = jnp.full((16,), 1.0, jnp.float32)
    for i in range(CH):
        for l in range(D // 16):
            ones_v[i, pl.ds(l * 16, 16)] = one
    _zero_fill(zbuf, 64, D)
    for t in range(ROWS_T // 64):
        pltpu.sync_copy(zbuf, acc_sh.at[pl.ds(s * ROWS_T + t * 64, 64)])
    pltpu.sync_copy(idx_hbm.at[c * NS + s], idx_v)
    plsc.subcore_barrier()

    def step(j, carry):
        pltpu.sync_copy(ones_v, acc_sh.at[idx_v.at[j]], add=True)
        return carry

    lax.fori_loop(0, K_DEG, step, 0)
    plsc.subcore_barrier()
    pltpu.sync_copy(acc_sh.at[pl.ds(s * ROWS_T, ROWS_T)],
                    out_hbm.at[c, pl.ds(s * ROWS_T, ROWS_T)])


@functools.partial(
    pl.kernel,
    out_type=jax.ShapeDtypeStruct((NC, N_ACC, D), jnp.float32),
    mesh=_mesh,
    scratch_types=[
        pltpu.VMEM((K_PUSH, CH), jnp.int32),
        pltpu.VMEM((K_PUSH, CH), jnp.int32),
        pltpu.VMEM((CH, D), jnp.float32),
        pltpu.VMEM((64, D), jnp.float32),
        pltpu.VMEM_SHARED((N_ACC, D), jnp.float32),
        pltpu.SemaphoreType.DMA,
    ],
)
def _push_kernel(src_hbm, dst_hbm, table_hbm, out_hbm,
                 src_v, dst_v, row_v, zbuf, acc_sh, sem):
    c = lax.axis_index("c")
    s = lax.axis_index("s")
    wid = c * NS + s
    _zero_fill(zbuf, 64, D)
    for t in range(ROWS_T // 64):
        pltpu.sync_copy(zbuf, acc_sh.at[pl.ds(s * ROWS_T + t * 64, 64)])
    pltpu.sync_copy(src_hbm.at[wid], src_v)
    pltpu.sync_copy(dst_hbm.at[wid], dst_v)
    plsc.subcore_barrier()

    def step(j, carry):
        pltpu.async_copy(table_hbm.at[src_v.at[j]], row_v, sem).wait()
        pltpu.sync_copy(row_v, acc_sh.at[dst_v.at[j]], add=True)
        return carry

    lax.fori_loop(0, K_PUSH, step, 0)
    plsc.subcore_barrier()
    pltpu.sync_copy(acc_sh.at[pl.ds(s * ROWS_T, ROWS_T)],
                    out_hbm.at[c, pl.ds(s * ROWS_T, ROWS_T)])


def _prep_body(all_ref, dego_ref, out_ref):
    rs = lax.rsqrt(jnp.maximum(dego_ref[...], 1.0))
    out_ref[...] = all_ref[...] * rs


BR = 1000


def _prep_call(all0, deg_out):
    return pl.pallas_call(
        _prep_body,
        out_shape=jax.ShapeDtypeStruct((N, D), jnp.float32),
        grid=(N // BR,),
        in_specs=[
            pl.BlockSpec((BR, D), lambda i: (i, 0)),
            pl.BlockSpec((BR, 1), lambda i: (i, 0)),
        ],
        out_specs=pl.BlockSpec((BR, D), lambda i: (i, 0)),
    )(all0, deg_out)


def _layer_body(acc_ref, all_ref, degi_ref, dego_ref,
                w1_ref, b1_ref, w2_ref, b2_ref,
                raw_ref, norm_ref, scaled_ref):
    nei = (acc_ref[0] + acc_ref[1]) * lax.rsqrt(jnp.maximum(degi_ref[...], 1.0))
    alle = all_ref[...]
    w1b = w1_ref[...] + b1_ref[...]
    w2b = w2_ref[...] + b2_ref[...]
    sum_e = jnp.dot(nei + alle, w1b, preferred_element_type=jnp.float32)
    sum_e = jnp.where(sum_e >= 0, sum_e, 0.2 * sum_e)
    bi = jnp.dot(nei * alle, w2b, preferred_element_type=jnp.float32)
    bi = jnp.where(bi >= 0, bi, 0.2 * bi)
    new = sum_e + bi
    nrm = jnp.sqrt(jnp.sum(new * new, axis=1, keepdims=True))
    raw_ref[...] = new
    norm_ref[...] = new / jnp.maximum(nrm, 1e-12)
    scaled_ref[...] = new * lax.rsqrt(jnp.maximum(dego_ref[...], 1.0))


def _layer_call(acc, alle, deg_in, deg_out, W1, b1, W2, b2):
    return pl.pallas_call(
        _layer_body,
        out_shape=(
            jax.ShapeDtypeStruct((N, D), jnp.float32),
            jax.ShapeDtypeStruct((N, D), jnp.float32),
            jax.ShapeDtypeStruct((N, D), jnp.float32),
        ),
        grid=(N // BR,),
        in_specs=[
            pl.BlockSpec((NC, BR, D), lambda i: (0, i, 0)),
            pl.BlockSpec((BR, D), lambda i: (i, 0)),
            pl.BlockSpec((BR, 1), lambda i: (i, 0)),
            pl.BlockSpec((BR, 1), lambda i: (i, 0)),
            pl.BlockSpec((D, D), lambda i: (0, 0)),
            pl.BlockSpec((1, D), lambda i: (0, 0)),
            pl.BlockSpec((D, D), lambda i: (0, 0)),
            pl.BlockSpec((1, D), lambda i: (0, 0)),
        ],
        out_specs=(
            pl.BlockSpec((BR, D), lambda i: (i, 0)),
            pl.BlockSpec((BR, D), lambda i: (i, 0)),
            pl.BlockSpec((BR, D), lambda i: (i, 0)),
        ),
    )(acc, alle, deg_in, deg_out, W1, b1, W2, b2)


def kernel(edge_index, embed_user, embed_item,
           W1_0, b1_0, W2_0, b2_0, W1_1, b1_1, W2_1, b2_1):
    src = edge_index[0].astype(jnp.int32)
    dst = edge_index[1].astype(jnp.int32)
    all0 = jnp.concatenate([embed_user, embed_item], axis=0)

    pad_d = jnp.full((NS * ET_PAD - E,), DUMP, jnp.int32)
    src_d = jnp.concatenate([src, pad_d]).reshape(NS, K_DEG, CH)
    dst_d = jnp.concatenate([dst, pad_d]).reshape(NS, K_DEG, CH)
    idx_deg = jnp.concatenate([src_d, dst_d], axis=0)
    deg = _deg_kernel(idx_deg)
    deg_out = deg[0, :N, :1]
    deg_in = deg[1, :N, :1]

    scaled0 = _prep_call(all0, deg_out)

    pad_s = jnp.zeros((NW * EW_PAD - E,), jnp.int32)
    pad_t = jnp.full((NW * EW_PAD - E,), DUMP, jnp.int32)
    srcp = jnp.concatenate([src, pad_s]).reshape(NW, K_PUSH, CH)
    dstp = jnp.concatenate([dst, pad_t]).reshape(NW, K_PUSH, CH)

    acc1 = _push_kernel(srcp, dstp, scaled0)
    raw1, norm1, scaled1 = _layer_call(acc1, all0, deg_in, deg_out,
                                       W1_0, b1_0, W2_0, b2_0)
    acc2 = _push_kernel(srcp, dstp, scaled1)
    _, norm2, _ = _layer_call(acc2, raw1, deg_in, deg_out,
                              W1_1, b1_1, W2_1, b2_1)

    final = jnp.concatenate([all0, norm1, norm2], axis=1)
    return (final[:N_USERS], final[N_USERS:])


# speedup vs baseline: 1.5798x; 1.0003x over previous
import functools

import jax
import jax.numpy as jnp
from jax import lax
from jax.experimental import pallas as pl
from jax.experimental.pallas import tpu as pltpu
from jax.experimental.pallas import tpu_sc as plsc

N_USERS = 6000
N_ITEMS = 4000
N = N_USERS + N_ITEMS
E = 320000
D = 128

NC = 2
NS = 16
NW = NC * NS

CH = 128
K_PUSH = -(-(E // NW) // CH)
EW_PAD = K_PUSH * CH
K_DEG = -(-(E // NS) // CH)
ET_PAD = K_DEG * CH

N_ACC = 10240
ROWS_T = N_ACC // NS
DUMP = N

_mesh = plsc.VectorSubcoreMesh(core_axis_name="c", subcore_axis_name="s",
                               num_cores=NC, num_subcores=NS)


def _zero_fill(zbuf, rows, width):
    zero = jnp.zeros((16,), jnp.float32)
    for i in range(rows):
        for l in range(width // 16):
            zbuf[i, pl.ds(l * 16, 16)] = zero


@functools.partial(
    pl.kernel,
    out_type=jax.ShapeDtypeStruct((NC, N_ACC, D), jnp.float32),
    mesh=_mesh,
    scratch_types=[
        pltpu.VMEM((K_DEG, CH), jnp.int32),
        pltpu.VMEM((CH, D), jnp.float32),
        pltpu.VMEM((64, D), jnp.float32),
        pltpu.VMEM_SHARED((N_ACC, D), jnp.float32),
    ],
)
def _deg_kernel(idx_hbm, out_hbm, idx_v, ones_v, zbuf, acc_sh):
    c = lax.axis_index("c")
    s = lax.axis_index("s")
    one = jnp.full((16,), 1.0, jnp.float32)
    for i in range(CH):
        for l in range(D // 16):
            ones_v[i, pl.ds(l * 16, 16)] = one
    _zero_fill(zbuf, 64, D)
    for t in range(ROWS_T // 64):
        pltpu.sync_copy(zbuf, acc_sh.at[pl.ds(s * ROWS_T + t * 64, 64)])
    pltpu.sync_copy(idx_hbm.at[c * NS + s], idx_v)
    plsc.subcore_barrier()

    def step(j, carry):
        pltpu.sync_copy(ones_v, acc_sh.at[idx_v.at[j]], add=True)
        return carry

    lax.fori_loop(0, K_DEG, step, 0)
    plsc.subcore_barrier()
    pltpu.sync_copy(acc_sh.at[pl.ds(s * ROWS_T, ROWS_T)],
                    out_hbm.at[c, pl.ds(s * ROWS_T, ROWS_T)])


@functools.partial(
    pl.kernel,
    out_type=jax.ShapeDtypeStruct((NC, N_ACC, D), jnp.float32),
    mesh=_mesh,
    scratch_types=[
        pltpu.VMEM((K_PUSH, CH), jnp.int32),
        pltpu.VMEM((K_PUSH, CH), jnp.int32),
        pltpu.VMEM((CH, D), jnp.float32),
        pltpu.VMEM((64, D), jnp.float32),
        pltpu.VMEM_SHARED((N_ACC, D), jnp.float32),
        pltpu.SemaphoreType.DMA,
    ],
)
def _push_kernel(src_hbm, dst_hbm, table_hbm, out_hbm,
                 src_v, dst_v, row_v, zbuf, acc_sh, sem):
    c = lax.axis_index("c")
    s = lax.axis_index("s")
    wid = c * NS + s
    _zero_fill(zbuf, 64, D)
    for t in range(ROWS_T // 64):
        pltpu.sync_copy(zbuf, acc_sh.at[pl.ds(s * ROWS_T + t * 64, 64)])
    pltpu.sync_copy(src_hbm.at[wid], src_v)
    pltpu.sync_copy(dst_hbm.at[wid], dst_v)
    plsc.subcore_barrier()

    def step(j, carry):
        pltpu.async_copy(table_hbm.at[src_v.at[j]], row_v, sem).wait()
        pltpu.sync_copy(row_v, acc_sh.at[dst_v.at[j]], add=True)
        return carry

    lax.fori_loop(0, K_PUSH, step, 0)
    plsc.subcore_barrier()
    pltpu.sync_copy(acc_sh.at[pl.ds(s * ROWS_T, ROWS_T)],
                    out_hbm.at[c, pl.ds(s * ROWS_T, ROWS_T)])


def _prep_body(all_ref, dego_ref, out_ref):
    rs = lax.rsqrt(jnp.maximum(dego_ref[...], 1.0))
    out_ref[...] = all_ref[...] * rs


BR = 1000


def _prep_call(all0, deg_out):
    return pl.pallas_call(
        _prep_body,
        out_shape=jax.ShapeDtypeStruct((N, D), jnp.float32),
        grid=(N // BR,),
        in_specs=[
            pl.BlockSpec((BR, D), lambda i: (i, 0)),
            pl.BlockSpec((BR, 1), lambda i: (i, 0)),
        ],
        out_specs=pl.BlockSpec((BR, D), lambda i: (i, 0)),
    )(all0, deg_out)


def _layer_body(acc_ref, all_ref, degi_ref, dego_ref,
                w1_ref, b1_ref, w2_ref, b2_ref,
                raw_ref, norm_ref, scaled_ref):
    nei = (acc_ref[0] + acc_ref[1]) * lax.rsqrt(jnp.maximum(degi_ref[...], 1.0))
    alle = all_ref[...]
    w1b = w1_ref[...] + b1_ref[...]
    w2b = w2_ref[...] + b2_ref[...]
    sum_e = jnp.dot(nei + alle, w1b, preferred_element_type=jnp.float32)
    sum_e = jnp.where(sum_e >= 0, sum_e, 0.2 * sum_e)
    bi = jnp.dot(nei * alle, w2b, preferred_element_type=jnp.float32)
    bi = jnp.where(bi >= 0, bi, 0.2 * bi)
    new = sum_e + bi
    nrm = jnp.sqrt(jnp.sum(new * new, axis=1, keepdims=True))
    raw_ref[...] = new
    norm_ref[...] = new / jnp.maximum(nrm, 1e-12)
    scaled_ref[...] = new * lax.rsqrt(jnp.maximum(dego_ref[...], 1.0))


def _layer_call(acc, alle, deg_in, deg_out, W1, b1, W2, b2):
    return pl.pallas_call(
        _layer_body,
        out_shape=(
            jax.ShapeDtypeStruct((N, D), jnp.float32),
            jax.ShapeDtypeStruct((N, D), jnp.float32),
            jax.ShapeDtypeStruct((N, D), jnp.float32),
        ),
        grid=(N // BR,),
        in_specs=[
            pl.BlockSpec((NC, BR, D), lambda i: (0, i, 0)),
            pl.BlockSpec((BR, D), lambda i: (i, 0)),
            pl.BlockSpec((BR, 1), lambda i: (i, 0)),
            pl.BlockSpec((BR, 1), lambda i: (i, 0)),
            pl.BlockSpec((D, D), lambda i: (0, 0)),
            pl.BlockSpec((1, D), lambda i: (0, 0)),
            pl.BlockSpec((D, D), lambda i: (0, 0)),
            pl.BlockSpec((1, D), lambda i: (0, 0)),
        ],
        out_specs=(
            pl.BlockSpec((BR, D), lambda i: (i, 0)),
            pl.BlockSpec((BR, D), lambda i: (i, 0)),
            pl.BlockSpec((BR, D), lambda i: (i, 0)),
        ),
    )(acc, alle, deg_in, deg_out, W1, b1, W2, b2)


def kernel(edge_index, embed_user, embed_item,
           W1_0, b1_0, W2_0, b2_0, W1_1, b1_1, W2_1, b2_1):
    src = edge_index[0].astype(jnp.int32)
    dst = edge_index[1].astype(jnp.int32)
    all0 = jnp.concatenate([embed_user, embed_item], axis=0)

    pad_d = jnp.full((NS * ET_PAD - E,), DUMP, jnp.int32)
    src_d = jnp.concatenate([src, pad_d]).reshape(NS, K_DEG, CH)
    dst_d = jnp.concatenate([dst, pad_d]).reshape(NS, K_DEG, CH)
    idx_deg = jnp.concatenate([src_d, dst_d], axis=0)
    deg = _deg_kernel(idx_deg)
    deg_out = deg[0, :N, :1]
    deg_in = deg[1, :N, :1]

    scaled0 = _prep_call(all0, deg_out)

    pad_s = jnp.zeros((NW * EW_PAD - E,), jnp.int32)
    pad_t = jnp.full((NW * EW_PAD - E,), DUMP, jnp.int32)
    srcp = jnp.concatenate([src, pad_s]).reshape(NW, K_PUSH, CH)
    dstp = jnp.concatenate([dst, pad_t]).reshape(NW, K_PUSH, CH)

    acc1 = _push_kernel(srcp, dstp, scaled0)
    raw1, norm1, scaled1 = _layer_call(acc1, all0, deg_in, deg_out,
                                       W1_0, b1_0, W2_0, b2_0)
    acc2 = _push_kernel(srcp, dstp, scaled1)
    _, norm2, _ = _layer_call(acc2, raw1, deg_in, deg_out,
                              W1_1, b1_1, W2_1, b2_1)

    final = jnp.concatenate([all0, norm1, norm2], axis=1)
    return (final[:N_USERS], final[N_USERS:])
